# Initial kernel scaffold; baseline (speedup 1.0000x reference)
import jax
import jax.numpy as jnp
from jax.experimental import pallas as pl


def kernel(x, conv_w, conv_b):
    raise NotImplementedError("write your pallas kernel here")


# speedup vs baseline: 6052.0761x; 6052.0761x over previous
import numpy as np
import jax
import jax.numpy as jnp
from jax.experimental import pallas as pl
from jax.experimental.pallas import tpu as pltpu

_C = 64
_H = 128
_W = 128
_PAD = 4
_NCFG = 9
_NRING = 16
_TH = 8


def _ring_offsets(rb, gb):
    gh, gw = gb
    h_prf = (rb + gh) * 2 + 1
    w_prf = (rb + gw) * 2 + 1
    xs = np.arange(-(rb + gh), rb + gh + 1)
    ys = np.arange(-(rb + gw), rb + gw + 1)
    px, py = np.meshgrid(xs, ys, indexing='ij')
    index_td = np.round(np.linspace(0, (rb + gw) * 2, 5)).astype(np.int64)
    index_lr = np.round(np.linspace(0, (rb + gh) * 2, 5)).astype(np.int64)[1:-1]

    def edges(m):
        t = m[0:rb][:, index_td]
        r = m[index_lr][:, w_prf - rb:w_prf]
        d = m[h_prf - rb:h_prf][:, index_td]
        l = m[index_lr][:, 0:rb]
        return np.concatenate([t.ravel(), r.ravel(), d.ravel(), l.ravel()])

    return list(zip(edges(px).tolist(), edges(py).tolist()))


_OFF = [_ring_offsets(1, (gh, gw)) for gh in (1, 2, 3) for gw in (1, 2, 3)]
_DIST = sorted(set(o for cfg in _OFF for o in cfg))
_GROUPS = []
for _n in range(_NRING):
    _d = {}
    for _g in range(_NCFG):
        _d.setdefault(_OFF[_g][_n], []).append(_g)
    _GROUPS.append(sorted(_d.items()))


def _body(xp_ref, w2_ref, b_ref, out_ref, win_ref, p_ref):
    t = pl.program_id(1)
    r0 = pl.multiple_of(t * _TH, _TH)
    win_ref[...] = xp_ref[0, :, pl.ds(r0, 2 * _TH), :]

    def shifted(dy, dx):
        return win_ref[:, _PAD + dy:_PAD + dy + _TH, _PAD + dx:_PAD + dx + _W]

    xc = shifted(0, 0)

    sims = [None] * _NCFG
    for (dy, dx) in _DIST:
        xs = shifted(dy, dx)
        s = jnp.sum(xc * xs, axis=0) * (1.0 / _C)
        sg = jax.nn.sigmoid(s)
        for g in range(_NCFG):
            if (dy, dx) in _OFF[g]:
                sims[g] = sg if sims[g] is None else sims[g] + sg
    sims = [s * (1.0 / _NRING) for s in sims]

    ranks = []
    for i in range(_NCFG):
        r = jnp.zeros((_TH, _W), jnp.float32)
        for j in range(_NCFG):
            if j == i:
                continue
            if j < i:
                r += (sims[j] <= sims[i]).astype(jnp.float32)
            else:
                r += (sims[j] < sims[i]).astype(jnp.float32)
        ranks.append(r)

    sortedv = []
    for rr in range(_NCFG):
        fr = float(rr)
        acc = jnp.where(ranks[0] == fr, sims[0], 0.0)
        for i in range(1, _NCFG):
            acc = jnp.where(ranks[i] == fr, sims[i], acc)
        sortedv.append(acc)
    diffs = [sortedv[rr + 1] - sortedv[rr] for rr in range(_NCFG - 1)]
    maxd = diffs[0]
    for rr in range(1, _NCFG - 1):
        maxd = jnp.maximum(maxd, diffs[rr])
    m = jnp.zeros((_TH, _W), jnp.float32)
    for rr in range(_NCFG - 2, -1, -1):
        m = jnp.where(diffs[rr] == maxd, float(rr), m)
    masks = [(ranks[g] == m).astype(jnp.float32)[None] for g in range(_NCFG)]

    for n in range(_NRING):
        acc = xc
        for (dy, dx), gs in _GROUPS[n]:
            coef = masks[gs[0]]
            for g in gs[1:]:
                coef = coef + masks[g]
            acc = acc - coef * shifted(dy, dx)
        p_ref[n * _C:(n + 1) * _C] = acc

    w2 = w2_ref[...]
    bias = b_ref[...]
    for h in range(_TH):
        out_ref[0, :, h, :] = jax.lax.dot_general(
            w2, p_ref[:, h, :], (((1,), (0,)), ((), ())),
            preferred_element_type=jnp.float32) + bias


def kernel(x, conv_w, conv_b):
    b, c, h, w = x.shape
    xp = jnp.pad(x, ((0, 0), (0, 0), (_PAD, _PAD), (_PAD, _PAD)))
    w2 = conv_w.reshape(_C, _C, _NRING).transpose(0, 2, 1).reshape(_C, _NRING * _C)
    bias = conv_b.reshape(_C, 1)
    nt = h // _TH
    hp, wp = h + 2 * _PAD, w + 2 * _PAD
    return pl.pallas_call(
        _body,
        grid=(b, nt),
        in_specs=[
            pl.BlockSpec((1, c, hp, wp), lambda bi, ti: (bi, 0, 0, 0)),
            pl.BlockSpec((_C, _NRING * _C), lambda bi, ti: (0, 0)),
            pl.BlockSpec((_C, 1), lambda bi, ti: (0, 0)),
        ],
        out_specs=pl.BlockSpec((1, _C, _TH, w), lambda bi, ti: (bi, 0, ti, 0)),
        out_shape=jax.ShapeDtypeStruct((b, _C, h, w), jnp.float32),
        scratch_shapes=[
            pltpu.VMEM((_C, 2 * _TH, wp), jnp.float32),
            pltpu.VMEM((_NRING * _C, _TH, w), jnp.float32),
        ],
        compiler_params=pltpu.CompilerParams(
            dimension_semantics=("arbitrary", "arbitrary"),
        ),
    )(xp, w2, bias)


# speedup vs baseline: 30355.8814x; 5.0158x over previous
import numpy as np
import jax
import jax.numpy as jnp
from jax.experimental import pallas as pl
from jax.experimental.pallas import tpu as pltpu

_C = 64
_H = 128
_W = 128
_PAD = 4
_NCFG = 9
_NRING = 16
_TH = 8


def _ring_offsets(rb, gb):
    gh, gw = gb
    h_prf = (rb + gh) * 2 + 1
    w_prf = (rb + gw) * 2 + 1
    xs = np.arange(-(rb + gh), rb + gh + 1)
    ys = np.arange(-(rb + gw), rb + gw + 1)
    px, py = np.meshgrid(xs, ys, indexing='ij')
    index_td = np.round(np.linspace(0, (rb + gw) * 2, 5)).astype(np.int64)
    index_lr = np.round(np.linspace(0, (rb + gh) * 2, 5)).astype(np.int64)[1:-1]

    def edges(m):
        t = m[0:rb][:, index_td]
        r = m[index_lr][:, w_prf - rb:w_prf]
        d = m[h_prf - rb:h_prf][:, index_td]
        l = m[index_lr][:, 0:rb]
        return np.concatenate([t.ravel(), r.ravel(), d.ravel(), l.ravel()])

    return list(zip(edges(px).tolist(), edges(py).tolist()))


_OFF = [_ring_offsets(1, (gh, gw)) for gh in (1, 2, 3) for gw in (1, 2, 3)]
_DIST = sorted(set(o for cfg in _OFF for o in cfg))
_GROUPS = []
for _n in range(_NRING):
    _d = {}
    for _g in range(_NCFG):
        _d.setdefault(_OFF[_g][_n], []).append(_g)
    _GROUPS.append(sorted(_d.items()))


def _body(xp_ref, w2_ref, b_ref, out_ref, win_ref, p_ref):
    t = pl.program_id(1)
    r0 = pl.multiple_of(t * _TH, _TH)
    for k in range(2 * _PAD + 1):
        win_ref[k * _C:(k + 1) * _C] = xp_ref[0, :, pl.ds(r0, 2 * _TH), k:k + _W]

    def shifted(dy, dx):
        k = dx + _PAD
        return win_ref[k * _C:(k + 1) * _C, _PAD + dy:_PAD + dy + _TH, :]

    xc = shifted(0, 0)

    sims = [None] * _NCFG
    for (dy, dx) in _DIST:
        xs = shifted(dy, dx)
        s = jnp.sum(xc * xs, axis=0) * (1.0 / _C)
        sg = jax.nn.sigmoid(s)
        for g in range(_NCFG):
            if (dy, dx) in _OFF[g]:
                sims[g] = sg if sims[g] is None else sims[g] + sg
    sims = [s * (1.0 / _NRING) for s in sims]

    ranks = []
    for i in range(_NCFG):
        r = jnp.zeros((_TH, _W), jnp.float32)
        for j in range(_NCFG):
            if j == i:
                continue
            if j < i:
                r += (sims[j] <= sims[i]).astype(jnp.float32)
            else:
                r += (sims[j] < sims[i]).astype(jnp.float32)
        ranks.append(r)

    sortedv = []
    for rr in range(_NCFG):
        fr = float(rr)
        acc = jnp.where(ranks[0] == fr, sims[0], 0.0)
        for i in range(1, _NCFG):
            acc = jnp.where(ranks[i] == fr, sims[i], acc)
        sortedv.append(acc)
    diffs = [sortedv[rr + 1] - sortedv[rr] for rr in range(_NCFG - 1)]
    maxd = diffs[0]
    for rr in range(1, _NCFG - 1):
        maxd = jnp.maximum(maxd, diffs[rr])
    m = jnp.zeros((_TH, _W), jnp.float32)
    for rr in range(_NCFG - 2, -1, -1):
        m = jnp.where(diffs[rr] == maxd, float(rr), m)
    masks = [(ranks[g] == m).astype(jnp.float32)[None] for g in range(_NCFG)]

    for n in range(_NRING):
        acc = xc
        for (dy, dx), gs in _GROUPS[n]:
            coef = masks[gs[0]]
            for g in gs[1:]:
                coef = coef + masks[g]
            acc = acc - coef * shifted(dy, dx)
        p_ref[n * _C:(n + 1) * _C] = acc.astype(jnp.bfloat16)

    w2 = w2_ref[...]
    bias = b_ref[...]
    for h in range(_TH):
        out_ref[0, :, h, :] = jax.lax.dot_general(
            w2, p_ref[:, h, :], (((1,), (0,)), ((), ())),
            preferred_element_type=jnp.float32) + bias


def kernel(x, conv_w, conv_b):
    b, c, h, w = x.shape
    xp = jnp.pad(x, ((0, 0), (0, 0), (_PAD, _PAD), (_PAD, _PAD)))
    w2 = conv_w.reshape(_C, _C, _NRING).transpose(0, 2, 1).reshape(
        _C, _NRING * _C).astype(jnp.bfloat16)
    bias = conv_b.reshape(_C, 1)
    nt = h // _TH
    hp, wp = h + 2 * _PAD, w + 2 * _PAD
    return pl.pallas_call(
        _body,
        grid=(b, nt),
        in_specs=[
            pl.BlockSpec((1, c, hp, wp), lambda bi, ti: (bi, 0, 0, 0)),
            pl.BlockSpec((_C, _NRING * _C), lambda bi, ti: (0, 0)),
            pl.BlockSpec((_C, 1), lambda bi, ti: (0, 0)),
        ],
        out_specs=pl.BlockSpec((1, _C, _TH, w), lambda bi, ti: (bi, 0, ti, 0)),
        out_shape=jax.ShapeDtypeStruct((b, _C, h, w), jnp.float32),
        scratch_shapes=[
            pltpu.VMEM(((2 * _PAD + 1) * _C, 2 * _TH, w), jnp.float32),
            pltpu.VMEM((_NRING * _C, _TH, w), jnp.bfloat16),
        ],
        compiler_params=pltpu.CompilerParams(
            dimension_semantics=("arbitrary", "arbitrary"),
        ),
    )(xp, w2, bias)
